# baseline (speedup 1.0000x reference)
import jax
import jax.numpy as jnp
from jax.experimental import pallas as pl
from jax.experimental.pallas import tpu as pltpu

B, C_IN, C_OUT, K = 8, 16, 32, 3
D_IN, H_IN, W_IN = 34, 66, 66
H_OUT, W_OUT = 64, 64
HW = H_IN * W_IN
HW_PAD = 4480
N_LANES = H_OUT * W_IN
N_J = 16
OFFS = tuple(kh * W_IN + kw for kh in range(K) for kw in range(K))


def _kernel(x_ref, w_ref, cb_ref, mask_ref, out_ref, yp_ref):
    j = pl.program_id(1)
    wm = w_ref[...]

    def im2col(d):
        base = pl.multiple_of(d * HW_PAD, 128)
        xs = x_ref[0, :, pl.ds(base, HW_PAD)]
        return jnp.concatenate(
            [xs[:, off:off + N_LANES] for off in OFFS], axis=0)

    def conv_pair(da, db):
        a = jnp.concatenate([im2col(da), im2col(db)], axis=1)
        y = jnp.dot(wm, a, preferred_element_type=jnp.float32)
        return y[:, :N_LANES], y[:, N_LANES:]

    @pl.when(j == 0)
    def _():
        ya, yb = conv_pair(0, 1)
        yp_ref[0] = ya
        yp_ref[1] = yb

    y2, y3 = conv_pair(2 * j + 2, 2 * j + 3)
    y0 = yp_ref[0, 0:32] + yp_ref[1, 32:64] + y2[64:96]
    y1 = yp_ref[1, 0:32] + y2[32:64] + y3[64:96]
    yp_ref[0] = y2
    yp_ref[1] = y3

    m = jnp.maximum(y0, y1) + cb_ref[...]
    ms1 = jnp.concatenate([m[:, 1:], m[:, :1]], axis=1)
    ma = jnp.maximum(m, ms1)
    ms66 = jnp.concatenate([ma[:, W_IN:], ma[:, :W_IN]], axis=1)
    mb = jnp.maximum(ma, ms66)
    csum = jnp.dot(mb.astype(jnp.bfloat16), mask_ref[...],
                   preferred_element_type=jnp.float32)

    @pl.when(j == 0)
    def _():
        out_ref[...] = jnp.zeros((1, C_OUT, 128), jnp.float32)

    out_ref[...] += csum.reshape(1, C_OUT, 128)


def _fmt_kernel(x_ref, o_ref):
    v = x_ref[0, 0].astype(jnp.bfloat16)
    o_ref[0, 0] = jnp.pad(v, ((0, 0), (0, HW_PAD - HW)))


@jax.jit
def kernel(x, conv_weight, conv_bias, bias):
    x6 = pl.pallas_call(
        _fmt_kernel,
        grid=(B, C_IN),
        in_specs=[pl.BlockSpec((1, 1, D_IN, HW), lambda b, c: (b, c, 0, 0))],
        out_specs=pl.BlockSpec((1, 1, D_IN, HW_PAD), lambda b, c: (b, c, 0, 0)),
        out_shape=jax.ShapeDtypeStruct((B, C_IN, D_IN, HW_PAD), jnp.bfloat16),
        compiler_params=pltpu.CompilerParams(
            dimension_semantics=("parallel", "arbitrary"),
        ),
    )(x.reshape(B, C_IN, D_IN, HW))
    x6 = x6.reshape(B, C_IN, D_IN * HW_PAD)
    wm = conv_weight.transpose(2, 0, 3, 4, 1).reshape(
        K * C_OUT, K * K * C_IN).astype(jnp.bfloat16)
    cb = conv_bias.reshape(C_OUT, 1)

    lane = jnp.arange(N_LANES, dtype=jnp.int32)
    h, w = lane // W_IN, lane % W_IN
    keep = (h % 2 == 0) & (w % 2 == 0) & (w < W_OUT)
    maskc = jnp.where(keep[:, None], jnp.ones((1,), jnp.bfloat16),
                      jnp.zeros((1,), jnp.bfloat16))
    maskc = jnp.broadcast_to(maskc, (N_LANES, 128))

    acc = pl.pallas_call(
        _kernel,
        grid=(B, N_J),
        in_specs=[
            pl.BlockSpec((1, C_IN, D_IN * HW_PAD), lambda b, j: (b, 0, 0)),
            pl.BlockSpec((K * C_OUT, K * K * C_IN), lambda b, j: (0, 0)),
            pl.BlockSpec((C_OUT, 1), lambda b, j: (0, 0)),
            pl.BlockSpec((N_LANES, 128), lambda b, j: (0, 0)),
        ],
        out_specs=pl.BlockSpec((1, C_OUT, 128), lambda b, j: (b, 0, 0)),
        out_shape=jax.ShapeDtypeStruct((B, C_OUT, 128), jnp.float32),
        scratch_shapes=[pltpu.VMEM((2, 3 * C_OUT, N_LANES), jnp.float32)],
        compiler_params=pltpu.CompilerParams(
            dimension_semantics=("parallel", "arbitrary"),
        ),
    )(x6, wm, cb, maskc)

    return (acc[:, :, 0].sum(axis=1) * 0.5 + bias.sum()).reshape(B, 1, 1, 1)


# speedup vs baseline: 1.0833x; 1.0833x over previous
import jax
import jax.numpy as jnp
from jax.experimental import pallas as pl
from jax.experimental.pallas import tpu as pltpu

B, C_IN, C_OUT, K = 8, 16, 32, 3
D_IN, H_IN, W_IN = 34, 66, 66
H_OUT, W_OUT = 64, 64
HW = H_IN * W_IN
HW_PAD = 4480
N_LANES = H_OUT * W_IN
N_J = 16
OFFS = tuple(kh * W_IN + kw for kh in range(K) for kw in range(K))


def _kernel(x_ref, w_ref, cb_ref, mask_ref, out_ref, yp_ref):
    j = pl.program_id(1)
    wm = w_ref[...]

    def im2col(d):
        base = pl.multiple_of(d * HW_PAD, 128)
        xs = x_ref[0, :, pl.ds(base, HW_PAD)]
        return jnp.concatenate(
            [xs[:, off:off + N_LANES] for off in OFFS], axis=0)

    def conv_pair(da, db):
        a = jnp.concatenate([im2col(da), im2col(db)], axis=1)
        y = jnp.dot(wm, a, preferred_element_type=jnp.float32)
        return y[:, :N_LANES], y[:, N_LANES:]

    @pl.when(j == 0)
    def _():
        ya, yb = conv_pair(0, 1)
        yp_ref[0] = ya
        yp_ref[1] = yb

    y2, y3 = conv_pair(2 * j + 2, 2 * j + 3)
    y0 = yp_ref[0, 0:32] + yp_ref[1, 32:64] + y2[64:96]
    y1 = yp_ref[1, 0:32] + y2[32:64] + y3[64:96]
    yp_ref[0] = y2
    yp_ref[1] = y3

    m = jnp.maximum(y0, y1) + cb_ref[...]
    ms1 = jnp.concatenate([m[:, 1:], m[:, :1]], axis=1)
    ma = jnp.maximum(m, ms1)
    ms66 = jnp.concatenate([ma[:, W_IN:], ma[:, :W_IN]], axis=1)
    mb = jnp.maximum(ma, ms66)
    csum = jnp.dot(mb.astype(jnp.bfloat16), mask_ref[...],
                   preferred_element_type=jnp.float32)

    @pl.when(j == 0)
    def _():
        out_ref[...] = jnp.zeros((1, C_OUT, 128), jnp.float32)

    out_ref[...] += csum.reshape(1, C_OUT, 128)


def _fmt_kernel(x_ref, o_ref):
    v = x_ref[0].astype(jnp.bfloat16)
    o_ref[0] = jnp.pad(v, ((0, 0), (0, 0), (0, HW_PAD - HW)))


@jax.jit
def kernel(x, conv_weight, conv_bias, bias):
    CG = 8
    x6 = pl.pallas_call(
        _fmt_kernel,
        grid=(B, C_IN // CG),
        in_specs=[pl.BlockSpec((1, CG, D_IN, HW), lambda b, c: (b, c, 0, 0))],
        out_specs=pl.BlockSpec((1, CG, D_IN, HW_PAD), lambda b, c: (b, c, 0, 0)),
        out_shape=jax.ShapeDtypeStruct((B, C_IN, D_IN, HW_PAD), jnp.bfloat16),
        compiler_params=pltpu.CompilerParams(
            dimension_semantics=("parallel", "arbitrary"),
        ),
    )(x.reshape(B, C_IN, D_IN, HW))
    x6 = x6.reshape(B, C_IN, D_IN * HW_PAD)
    wm = conv_weight.transpose(2, 0, 3, 4, 1).reshape(
        K * C_OUT, K * K * C_IN).astype(jnp.bfloat16)
    cb = conv_bias.reshape(C_OUT, 1)

    lane = jnp.arange(N_LANES, dtype=jnp.int32)
    h, w = lane // W_IN, lane % W_IN
    keep = (h % 2 == 0) & (w % 2 == 0) & (w < W_OUT)
    maskc = jnp.where(keep[:, None], jnp.ones((1,), jnp.bfloat16),
                      jnp.zeros((1,), jnp.bfloat16))
    maskc = jnp.broadcast_to(maskc, (N_LANES, 128))

    acc = pl.pallas_call(
        _kernel,
        grid=(B, N_J),
        in_specs=[
            pl.BlockSpec((1, C_IN, D_IN * HW_PAD), lambda b, j: (b, 0, 0)),
            pl.BlockSpec((K * C_OUT, K * K * C_IN), lambda b, j: (0, 0)),
            pl.BlockSpec((C_OUT, 1), lambda b, j: (0, 0)),
            pl.BlockSpec((N_LANES, 128), lambda b, j: (0, 0)),
        ],
        out_specs=pl.BlockSpec((1, C_OUT, 128), lambda b, j: (b, 0, 0)),
        out_shape=jax.ShapeDtypeStruct((B, C_OUT, 128), jnp.float32),
        scratch_shapes=[pltpu.VMEM((2, 3 * C_OUT, N_LANES), jnp.float32)],
        compiler_params=pltpu.CompilerParams(
            dimension_semantics=("parallel", "arbitrary"),
        ),
    )(x6, wm, cb, maskc)

    return (acc[:, :, 0].sum(axis=1) * 0.5 + bias.sum()).reshape(B, 1, 1, 1)


# speedup vs baseline: 1.2745x; 1.1765x over previous
import jax
import jax.numpy as jnp
from jax.experimental import pallas as pl
from jax.experimental.pallas import tpu as pltpu

B, C_IN, C_OUT, K = 8, 16, 32, 3
D_IN, H_IN, W_IN = 34, 66, 66
H_OUT, W_OUT = 64, 64
HW = H_IN * W_IN
HW_PAD = 4480
N_LANES = H_OUT * W_IN
N_J = 16
OFFS = tuple(kh * W_IN + kw for kh in range(K) for kw in range(K))


def _kernel(x0_ref, x1_ref, x2_ref, x3_ref, w_ref, cb_ref, mask_ref,
            out_ref, yp_ref):
    j = pl.program_id(1)
    wm = w_ref[...]

    def im2col(ref):
        xs = ref[0, 0]
        return jnp.concatenate(
            [xs[:, off:off + N_LANES] for off in OFFS], axis=0)

    def conv_pair(ra, rb):
        a = jnp.concatenate([im2col(ra), im2col(rb)], axis=1)
        y = jnp.dot(wm, a, preferred_element_type=jnp.float32)
        return y[:, :N_LANES], y[:, N_LANES:]

    @pl.when(j == 0)
    def _():
        ya, yb = conv_pair(x0_ref, x1_ref)
        yp_ref[0] = ya
        yp_ref[1] = yb

    y2, y3 = conv_pair(x2_ref, x3_ref)
    y0 = yp_ref[0, 0:32] + yp_ref[1, 32:64] + y2[64:96]
    y1 = yp_ref[1, 0:32] + y2[32:64] + y3[64:96]
    yp_ref[0] = y2
    yp_ref[1] = y3

    m = jnp.maximum(y0, y1) + cb_ref[...]
    ms1 = jnp.concatenate([m[:, 1:], m[:, :1]], axis=1)
    ma = jnp.maximum(m, ms1)
    ms66 = jnp.concatenate([ma[:, W_IN:], ma[:, :W_IN]], axis=1)
    mb = jnp.maximum(ma, ms66)
    csum = jnp.dot(mb.astype(jnp.bfloat16), mask_ref[...],
                   preferred_element_type=jnp.float32)

    @pl.when(j == 0)
    def _():
        out_ref[...] = jnp.zeros((1, C_OUT, 128), jnp.float32)

    out_ref[...] += csum.reshape(1, C_OUT, 128)


@jax.jit
def kernel(x, conv_weight, conv_bias, bias):
    x5 = jnp.pad(
        x.astype(jnp.bfloat16).reshape(B, C_IN, D_IN, HW).transpose(0, 2, 1, 3),
        ((0, 0), (0, 0), (0, 0), (0, HW_PAD - HW)))
    wm = conv_weight.transpose(2, 0, 3, 4, 1).reshape(
        K * C_OUT, K * K * C_IN).astype(jnp.bfloat16)
    cb = conv_bias.reshape(C_OUT, 1)

    lane = jnp.arange(N_LANES, dtype=jnp.int32)
    h, w = lane // W_IN, lane % W_IN
    keep = (h % 2 == 0) & (w % 2 == 0) & (w < W_OUT)
    maskc = jnp.where(keep[:, None], jnp.ones((1,), jnp.bfloat16),
                      jnp.zeros((1,), jnp.bfloat16))
    maskc = jnp.broadcast_to(maskc, (N_LANES, 128))

    slab_spec = [
        pl.BlockSpec((1, 1, C_IN, HW_PAD), lambda b, j: (b, 0, 0, 0)),
        pl.BlockSpec((1, 1, C_IN, HW_PAD), lambda b, j: (b, 1, 0, 0)),
        pl.BlockSpec((1, 1, C_IN, HW_PAD), lambda b, j: (b, 2 * j + 2, 0, 0)),
        pl.BlockSpec((1, 1, C_IN, HW_PAD), lambda b, j: (b, 2 * j + 3, 0, 0)),
    ]
    acc = pl.pallas_call(
        _kernel,
        grid=(B, N_J),
        in_specs=slab_spec + [
            pl.BlockSpec((K * C_OUT, K * K * C_IN), lambda b, j: (0, 0)),
            pl.BlockSpec((C_OUT, 1), lambda b, j: (0, 0)),
            pl.BlockSpec((N_LANES, 128), lambda b, j: (0, 0)),
        ],
        out_specs=pl.BlockSpec((1, C_OUT, 128), lambda b, j: (b, 0, 0)),
        out_shape=jax.ShapeDtypeStruct((B, C_OUT, 128), jnp.float32),
        scratch_shapes=[pltpu.VMEM((2, 3 * C_OUT, N_LANES), jnp.float32)],
        compiler_params=pltpu.CompilerParams(
            dimension_semantics=("parallel", "arbitrary"),
        ),
    )(x5, x5, x5, x5, wm, cb, maskc)

    return (acc[:, :, 0].sum(axis=1) * 0.5 + bias.sum()).reshape(B, 1, 1, 1)


# speedup vs baseline: 1.3094x; 1.0273x over previous
import jax
import jax.numpy as jnp
from jax.experimental import pallas as pl
from jax.experimental.pallas import tpu as pltpu

B, C_IN, C_OUT, K = 8, 16, 32, 3
D_IN, H_IN, W_IN = 34, 66, 66
H_OUT, W_OUT = 64, 64
H_PAD, W_STR = 67, 70
HW = H_PAD * W_STR
N_LANES = H_OUT * W_STR
N_J = 16
OFFS = tuple(kh * W_STR + kw for kh in range(K) for kw in range(K))


def _fmt_kernel(x_ref, o_ref):
    for d in range(D_IN):
        v = x_ref[0, :, d].astype(jnp.bfloat16)
        o_ref[0, d] = jnp.pad(
            v, ((0, 0), (0, H_PAD - H_IN), (0, W_STR - W_IN)))


def _kernel(x0_ref, x1_ref, x2_ref, x3_ref, w_ref, cb_ref, mask_ref,
            out_ref, yp_ref):
    j = pl.program_id(1)
    wm = w_ref[...]

    def im2col(ref):
        xs = ref[0, 0]
        return jnp.concatenate(
            [xs[:, off:off + N_LANES] for off in OFFS], axis=0)

    def conv_pair(ra, rb):
        a = jnp.concatenate([im2col(ra), im2col(rb)], axis=1)
        y = jnp.dot(wm, a, preferred_element_type=jnp.float32)
        return y[:, :N_LANES], y[:, N_LANES:]

    @pl.when(j == 0)
    def _():
        ya, yb = conv_pair(x0_ref, x1_ref)
        yp_ref[0] = ya
        yp_ref[1] = yb

    y2, y3 = conv_pair(x2_ref, x3_ref)
    y0 = yp_ref[0, 0:32] + yp_ref[1, 32:64] + y2[64:96]
    y1 = yp_ref[1, 0:32] + y2[32:64] + y3[64:96]
    yp_ref[0] = y2
    yp_ref[1] = y3

    m = jnp.maximum(y0, y1) + cb_ref[...]
    ms1 = jnp.concatenate([m[:, 1:], m[:, :1]], axis=1)
    ma = jnp.maximum(m, ms1)
    msw = jnp.concatenate([ma[:, W_STR:], ma[:, :W_STR]], axis=1)
    mb = jnp.maximum(ma, msw)
    csum = jnp.dot(mb.astype(jnp.bfloat16), mask_ref[...],
                   preferred_element_type=jnp.float32)

    @pl.when(j == 0)
    def _():
        out_ref[...] = jnp.zeros((1, C_OUT, 128), jnp.float32)

    out_ref[...] += csum.reshape(1, C_OUT, 128)


@jax.jit
def kernel(x, conv_weight, conv_bias, bias):
    xt = pl.pallas_call(
        _fmt_kernel,
        grid=(B,),
        in_specs=[pl.BlockSpec((1, C_IN, D_IN, H_IN, W_IN),
                               lambda b: (b, 0, 0, 0, 0))],
        out_specs=pl.BlockSpec((1, D_IN, C_IN, H_PAD, W_STR),
                               lambda b: (b, 0, 0, 0, 0)),
        out_shape=jax.ShapeDtypeStruct((B, D_IN, C_IN, H_PAD, W_STR),
                                       jnp.bfloat16),
        compiler_params=pltpu.CompilerParams(
            dimension_semantics=("parallel",),
        ),
    )(x)
    x5 = xt.reshape(B, D_IN, C_IN, HW)

    wm = conv_weight.transpose(2, 0, 3, 4, 1).reshape(
        K * C_OUT, K * K * C_IN).astype(jnp.bfloat16)
    cb = conv_bias.reshape(C_OUT, 1)

    lane = jnp.arange(N_LANES, dtype=jnp.int32)
    h, w = lane // W_STR, lane % W_STR
    keep = (h % 2 == 0) & (w % 2 == 0) & (w < W_OUT)
    maskc = jnp.where(keep[:, None], jnp.ones((1,), jnp.bfloat16),
                      jnp.zeros((1,), jnp.bfloat16))
    maskc = jnp.broadcast_to(maskc, (N_LANES, 128))

    slab_spec = [
        pl.BlockSpec((1, 1, C_IN, HW), lambda b, j: (b, 0, 0, 0)),
        pl.BlockSpec((1, 1, C_IN, HW), lambda b, j: (b, 1, 0, 0)),
        pl.BlockSpec((1, 1, C_IN, HW), lambda b, j: (b, 2 * j + 2, 0, 0)),
        pl.BlockSpec((1, 1, C_IN, HW), lambda b, j: (b, 2 * j + 3, 0, 0)),
    ]
    acc = pl.pallas_call(
        _kernel,
        grid=(B, N_J),
        in_specs=slab_spec + [
            pl.BlockSpec((K * C_OUT, K * K * C_IN), lambda b, j: (0, 0)),
            pl.BlockSpec((C_OUT, 1), lambda b, j: (0, 0)),
            pl.BlockSpec((N_LANES, 128), lambda b, j: (0, 0)),
        ],
        out_specs=pl.BlockSpec((1, C_OUT, 128), lambda b, j: (b, 0, 0)),
        out_shape=jax.ShapeDtypeStruct((B, C_OUT, 128), jnp.float32),
        scratch_shapes=[pltpu.VMEM((2, 3 * C_OUT, N_LANES), jnp.float32)],
        compiler_params=pltpu.CompilerParams(
            dimension_semantics=("parallel", "arbitrary"),
        ),
    )(x5, x5, x5, x5, wm, cb, maskc)

    return (acc[:, :, 0].sum(axis=1) * 0.5 + bias.sum()).reshape(B, 1, 1, 1)
